# baseline (speedup 1.0000x reference)
import functools
import jax
import jax.numpy as jnp
from jax import lax
from jax.experimental import pallas as pl
from jax.experimental.pallas import tpu as pltpu
from jax.experimental.pallas import tpu_sc as plsc

_IOU = 0.5
_CONF = 0.5
_MAXDET = 100
_N = 20000
_R = 160
_C = 128
_NP = _R * _C
_NCLS = 20
_NTILE = 16
_CHUNK = _NP // _NTILE
_NV = _CHUNK // 16


def _decode_body(cls_ref, box_ref, anc_ref, f_ref):
    best_s = jax.nn.sigmoid(cls_ref[0, 0])
    best_c = jnp.zeros((_R, _C), jnp.float32)
    for c in range(1, _NCLS):
        s = jax.nn.sigmoid(cls_ref[0, c])
        m = s > best_s
        best_s = jnp.where(m, s, best_s)
        best_c = jnp.where(m, jnp.float32(c), best_c)
    conf = best_s
    scores = jnp.where(conf > _CONF, conf, -1.0)
    ax = anc_ref[0]
    ay = anc_ref[1]
    aw = anc_ref[2]
    ah = anc_ref[3]
    d0 = box_ref[0, 0] * jnp.float32(0.1)
    d1 = box_ref[0, 1] * jnp.float32(0.1)
    d2 = box_ref[0, 2] * jnp.float32(0.2)
    d3 = box_ref[0, 3] * jnp.float32(0.2)
    bx = d0 * aw + ax
    by = d1 * ah + ay
    bw = jnp.exp(d2) * aw
    bh = jnp.exp(d3) * ah
    off = best_c * jnp.float32(10000.0)
    x1 = bx + off
    y1 = by + off
    x2 = (bx + bw) + off
    y2 = (by + bh) + off
    areas = (x2 - x1) * (y2 - y1)
    f_ref[0] = jnp.stack(
        [scores, x1, y1, x2, y2, areas, bx, by, bw, bh, best_c, conf],
        axis=0)


def _tc_decode(cls_t, box_t, anc_t, B):
    return pl.pallas_call(
        _decode_body,
        grid=(B,),
        in_specs=[
            pl.BlockSpec((1, _NCLS, _R, _C), lambda b: (b, 0, 0, 0)),
            pl.BlockSpec((1, 4, _R, _C), lambda b: (b, 0, 0, 0)),
            pl.BlockSpec((4, _R, _C), lambda b: (0, 0, 0)),
        ],
        out_specs=pl.BlockSpec((1, 12, _R, _C), lambda b: (b, 0, 0, 0)),
        out_shape=jax.ShapeDtypeStruct((B, 12, _R, _C), jnp.float32),
    )(cls_t, box_t, anc_t)


def _sc_nms(fields):
    mesh = plsc.VectorSubcoreMesh(core_axis_name="c", subcore_axis_name="s")

    @functools.partial(
        pl.kernel,
        mesh=mesh,
        out_type=jax.ShapeDtypeStruct((2, _MAXDET * 16), jnp.float32),
        scratch_types=[
            pltpu.VMEM((12, _CHUNK), jnp.float32),
            pltpu.VMEM((_CHUNK,), jnp.float32),
            pltpu.VMEM((16,), jnp.float32),
            pltpu.VMEM((16 * _NTILE,), jnp.float32),
            pltpu.VMEM((16,), jnp.float32),
            pltpu.VMEM((_MAXDET * 16,), jnp.float32),
            pltpu.VMEM_SHARED((16 * _NTILE + 16,), jnp.float32),
        ],
    )
    def k(fields_hbm, out_hbm, fld, sc, rec, candv, wloc, win, shared):
        core = lax.axis_index("c")
        sid = lax.axis_index("s")
        base = sid * _CHUNK
        lanei = jnp.arange(16, dtype=jnp.int32)

        def bmaxf(v):
            for kk in (1, 2, 4, 8):
                p = jnp.bitwise_xor(lanei, jnp.int32(kk))
                v = jnp.maximum(v, v.at[p].get(mode="promise_in_bounds"))
            return v

        def bmin(v):
            for kk in (1, 2, 4, 8):
                p = jnp.bitwise_xor(lanei, jnp.int32(kk))
                v = jnp.minimum(v, v.at[p].get(mode="promise_in_bounds"))
            return v

        for f in range(12):
            off = (core * 12 + f) * _NP + base
            pltpu.sync_copy(fields_hbm.at[pl.ds(off, _CHUNK)], fld.at[f])

        bestv0 = fld[0, pl.ds(0, 16)]
        bestj0 = jnp.zeros((16,), jnp.int32)
        sc[pl.ds(0, 16)] = bestv0
        for j in range(1, _NV):
            v = fld[0, pl.ds(j * 16, 16)]
            sc[pl.ds(j * 16, 16)] = v
            m = v > bestv0
            bestv0 = jnp.where(m, v, bestv0)
            bestj0 = jnp.where(m, jnp.int32(j), bestj0)

        def body(i, carry):
            bestv, bestj = carry
            bvv = bmaxf(bestv)
            lidx = bestj * 16 + lanei
            lminv = bmin(jnp.where(bestv == bvv, lidx, jnp.int32(2 ** 30)))
            lmin_s = lminv[0]
            jstar = lax.shift_right_logical(lmin_s, 4)
            lsel = jnp.bitwise_and(lminv, jnp.int32(15))

            gidxf = (base + lmin_s).astype(jnp.float32)
            r = jnp.where(lanei == 0, bvv, 0.0)
            r = jnp.where(lanei == 1, gidxf, r)
            for f in range(1, 12):
                row = fld[f, pl.ds(jstar * 16, 16)]
                g = row.at[lsel].get(mode="promise_in_bounds")
                r = jnp.where(lanei == 1 + f, g, r)
            rec[...] = r
            pltpu.sync_copy(rec, shared.at[pl.ds(sid * 16, 16)])
            plsc.subcore_barrier()

            pltpu.sync_copy(shared.at[pl.ds(0, 16 * _NTILE)], candv)
            vals = jnp.zeros((16,), jnp.float32)
            idxs = jnp.zeros((16,), jnp.float32)
            zi = lanei * 0
            for t in range(_NTILE):
                row = candv[pl.ds(t * 16, 16)]
                b0 = row.at[zi].get(mode="promise_in_bounds")
                b1 = row.at[zi + 1].get(mode="promise_in_bounds")
                vals = jnp.where(lanei == t, b0, vals)
                idxs = jnp.where(lanei == t, b1, idxs)
            bvg = bmaxf(vals)
            gminv = bmin(jnp.where(vals == bvg, idxs, jnp.float32(3e7)))
            wtv = bmin(jnp.where((vals == bvg) & (idxs == gminv),
                                 lanei, jnp.int32(16)))
            wrec = candv[pl.ds(wtv[0] * 16, 16)]

            @pl.when(sid == 0)
            def _():
                vv = bvg > 0.0
                outlane = (lanei >= 7) & (lanei <= 12)
                masked = jnp.where(
                    outlane, jnp.where(vv, wrec, jnp.float32(-1.0)), wrec)
                win[pl.ds(i * 16, 16)] = masked

            ex1 = wrec[2]
            ey1 = wrec[3]
            ex2 = wrec[4]
            ey2 = wrec[5]
            ear = wrec[6]
            nbv = jnp.full((16,), -2.0, jnp.float32)
            nbj = jnp.zeros((16,), jnp.int32)
            for j in range(_NV):
                s = pl.ds(j * 16, 16)
                a1 = fld[1, s]
                a2 = fld[2, s]
                a3 = fld[3, s]
                a4 = fld[4, s]
                xx1 = jnp.maximum(ex1, a1)
                yy1 = jnp.maximum(ey1, a2)
                xx2 = jnp.minimum(ex2, a3)
                yy2 = jnp.minimum(ey2, a4)
                inter = (jnp.maximum(xx2 - xx1, 0.0)
                         * jnp.maximum(yy2 - yy1, 0.0))
                areav = (a3 - a1) * (a4 - a2)
                iou = inter / (ear + areav - inter + jnp.float32(1e-8))
                nsc = jnp.where(iou >= _IOU, -1.0, sc[s])
                sc[s] = nsc
                m = nsc > nbv
                nbv = jnp.where(m, nsc, nbv)
                nbj = jnp.where(m, jnp.int32(j), nbj)
            return nbv, nbj

        lax.fori_loop(0, _MAXDET, body, (bestv0, bestj0))

        @pl.when(sid == 0)
        def _():
            pltpu.sync_copy(win, out_hbm.at[core])

    return k(fields)


def kernel(images, box_preds, cls_preds, anchors):
    del images
    B = box_preds.shape[0]
    padn = _NP - _N
    cls_t = jnp.transpose(cls_preds, (0, 2, 1))
    cls_t = jnp.pad(cls_t, ((0, 0), (0, 0), (0, padn)), constant_values=-1e9)
    cls_t = cls_t.reshape(B, _NCLS, _R, _C)
    box_t = jnp.transpose(box_preds, (0, 2, 1))
    box_t = jnp.pad(box_t, ((0, 0), (0, 0), (0, padn))).reshape(B, 4, _R, _C)
    anc_t = jnp.pad(anchors.T, ((0, 0), (0, padn))).reshape(4, _R, _C)

    fields = _tc_decode(cls_t, box_t, anc_t, B).reshape(B * 12 * _NP)
    out = _sc_nms(fields).reshape(B, _MAXDET, 16)

    out_boxes = out[:, :, 7:11]
    out_classes = out[:, :, 11]
    out_conf = out[:, :, 12]
    return out_boxes, out_classes, out_conf


# speedup vs baseline: 1.0464x; 1.0464x over previous
import functools
import jax
import jax.numpy as jnp
from jax import lax
from jax.experimental import pallas as pl
from jax.experimental.pallas import tpu as pltpu
from jax.experimental.pallas import tpu_sc as plsc

_IOU = 0.5
_CONF = 0.5
_MAXDET = 100
_N = 20000
_R = 160
_C = 128
_NP = _R * _C
_NCLS = 20
_NTILE = 16
_CHUNK = _NP // _NTILE
_NV = _CHUNK // 16


def _decode_body(cls_ref, box_ref, anc_ref, f_ref):
    best_s = jax.nn.sigmoid(cls_ref[0, 0])
    best_c = jnp.zeros((_R, _C), jnp.float32)
    for c in range(1, _NCLS):
        s = jax.nn.sigmoid(cls_ref[0, c])
        m = s > best_s
        best_s = jnp.where(m, s, best_s)
        best_c = jnp.where(m, jnp.float32(c), best_c)
    conf = best_s
    scores = jnp.where(conf > _CONF, conf, -1.0)
    ax = anc_ref[0]
    ay = anc_ref[1]
    aw = anc_ref[2]
    ah = anc_ref[3]
    d0 = box_ref[0, 0] * jnp.float32(0.1)
    d1 = box_ref[0, 1] * jnp.float32(0.1)
    d2 = box_ref[0, 2] * jnp.float32(0.2)
    d3 = box_ref[0, 3] * jnp.float32(0.2)
    bx = d0 * aw + ax
    by = d1 * ah + ay
    bw = jnp.exp(d2) * aw
    bh = jnp.exp(d3) * ah
    off = best_c * jnp.float32(10000.0)
    x1 = bx + off
    y1 = by + off
    x2 = (bx + bw) + off
    y2 = (by + bh) + off
    areas = (x2 - x1) * (y2 - y1)
    f_ref[0] = jnp.stack(
        [scores, x1, y1, x2, y2, areas, bx, by, bw, bh, best_c, conf],
        axis=0)


def _tc_decode(cls_t, box_t, anc_t, B):
    return pl.pallas_call(
        _decode_body,
        grid=(B,),
        in_specs=[
            pl.BlockSpec((1, _NCLS, _R, _C), lambda b: (b, 0, 0, 0)),
            pl.BlockSpec((1, 4, _R, _C), lambda b: (b, 0, 0, 0)),
            pl.BlockSpec((4, _R, _C), lambda b: (0, 0, 0)),
        ],
        out_specs=pl.BlockSpec((1, 12, _R, _C), lambda b: (b, 0, 0, 0)),
        out_shape=jax.ShapeDtypeStruct((B, 12, _R, _C), jnp.float32),
    )(cls_t, box_t, anc_t)


def _sc_nms(fields):
    mesh = plsc.VectorSubcoreMesh(core_axis_name="c", subcore_axis_name="s")

    @functools.partial(
        pl.kernel,
        mesh=mesh,
        out_type=jax.ShapeDtypeStruct((2, _MAXDET * 16), jnp.float32),
        scratch_types=[
            pltpu.VMEM((12, _CHUNK), jnp.float32),
            pltpu.VMEM((_CHUNK,), jnp.float32),
            pltpu.VMEM((16,), jnp.float32),
            pltpu.VMEM((16 * _NTILE,), jnp.float32),
            pltpu.VMEM((16,), jnp.float32),
            pltpu.VMEM((_MAXDET * 16,), jnp.float32),
            pltpu.VMEM_SHARED((16 * _NTILE + 16,), jnp.float32),
        ],
    )
    def k(fields_hbm, out_hbm, fld, sc, rec, candv, wloc, win, shared):
        core = lax.axis_index("c")
        sid = lax.axis_index("s")
        base = sid * _CHUNK
        lanei = jnp.arange(16, dtype=jnp.int32)

        def bmaxf(v):
            for kk in (1, 2, 4, 8):
                p = jnp.bitwise_xor(lanei, jnp.int32(kk))
                v = jnp.maximum(v, v.at[p].get(mode="promise_in_bounds"))
            return v

        def bmin(v):
            for kk in (1, 2, 4, 8):
                p = jnp.bitwise_xor(lanei, jnp.int32(kk))
                v = jnp.minimum(v, v.at[p].get(mode="promise_in_bounds"))
            return v

        for f in range(12):
            off = (core * 12 + f) * _NP + base
            pltpu.sync_copy(fields_hbm.at[pl.ds(off, _CHUNK)], fld.at[f])

        bestv0 = fld[0, pl.ds(0, 16)]
        bestj0 = jnp.zeros((16,), jnp.int32)
        sc[pl.ds(0, 16)] = bestv0
        for j in range(1, _NV):
            v = fld[0, pl.ds(j * 16, 16)]
            sc[pl.ds(j * 16, 16)] = v
            m = v > bestv0
            bestv0 = jnp.where(m, v, bestv0)
            bestj0 = jnp.where(m, jnp.int32(j), bestj0)

        def body(i, carry):
            bestv, bestj = carry
            bvv = bmaxf(bestv)
            lidx = bestj * 16 + lanei
            lminv = bmin(jnp.where(bestv == bvv, lidx, jnp.int32(2 ** 30)))
            lmin_s = lminv[0]
            jstar = lax.shift_right_logical(lmin_s, 4)
            lsel = jnp.bitwise_and(lminv, jnp.int32(15))

            gidxf = (base + lmin_s).astype(jnp.float32)
            r = jnp.where(lanei == 0, bvv, 0.0)
            r = jnp.where(lanei == 1, gidxf, r)
            for f in range(1, 12):
                row = fld[f, pl.ds(jstar * 16, 16)]
                g = row.at[lsel].get(mode="promise_in_bounds")
                r = jnp.where(lanei == 1 + f, g, r)
            rec[...] = r
            pltpu.sync_copy(rec, shared.at[pl.ds(sid * 16, 16)])
            plsc.subcore_barrier()

            pltpu.sync_copy(shared.at[pl.ds(0, 16 * _NTILE)], candv)
            vals = jnp.zeros((16,), jnp.float32)
            idxs = jnp.zeros((16,), jnp.float32)
            zi = lanei * 0
            for t in range(_NTILE):
                row = candv[pl.ds(t * 16, 16)]
                b0 = row.at[zi].get(mode="promise_in_bounds")
                b1 = row.at[zi + 1].get(mode="promise_in_bounds")
                vals = jnp.where(lanei == t, b0, vals)
                idxs = jnp.where(lanei == t, b1, idxs)
            bvg = bmaxf(vals)
            gminv = bmin(jnp.where(vals == bvg, idxs, jnp.float32(3e7)))
            wtv = bmin(jnp.where((vals == bvg) & (idxs == gminv),
                                 lanei, jnp.int32(16)))
            wrec = candv[pl.ds(wtv[0] * 16, 16)]

            @pl.when(sid == 0)
            def _():
                vv = bvg > 0.0
                outlane = (lanei >= 7) & (lanei <= 12)
                masked = jnp.where(
                    outlane, jnp.where(vv, wrec, jnp.float32(-1.0)), wrec)
                win[pl.ds(i * 16, 16)] = masked

            ex1 = wrec[2]
            ey1 = wrec[3]
            ex2 = wrec[4]
            ey2 = wrec[5]
            ear = wrec[6]
            nbv = jnp.full((16,), -2.0, jnp.float32)
            nbj = jnp.zeros((16,), jnp.int32)
            for j in range(_NV):
                s = pl.ds(j * 16, 16)
                xx1 = jnp.maximum(ex1, fld[1, s])
                yy1 = jnp.maximum(ey1, fld[2, s])
                xx2 = jnp.minimum(ex2, fld[3, s])
                yy2 = jnp.minimum(ey2, fld[4, s])
                inter = (jnp.maximum(xx2 - xx1, 0.0)
                         * jnp.maximum(yy2 - yy1, 0.0))
                iou = inter / (ear + fld[5, s] - inter + jnp.float32(1e-8))
                nsc = jnp.where(iou >= _IOU, -1.0, sc[s])
                sc[s] = nsc
                m = nsc > nbv
                nbv = jnp.where(m, nsc, nbv)
                nbj = jnp.where(m, jnp.int32(j), nbj)
            return nbv, nbj

        lax.fori_loop(0, _MAXDET, body, (bestv0, bestj0))

        @pl.when(sid == 0)
        def _():
            pltpu.sync_copy(win, out_hbm.at[core])

    return k(fields)


def kernel(images, box_preds, cls_preds, anchors):
    del images
    B = box_preds.shape[0]
    padn = _NP - _N
    cls_t = jnp.transpose(cls_preds, (0, 2, 1))
    cls_t = jnp.pad(cls_t, ((0, 0), (0, 0), (0, padn)), constant_values=-1e9)
    cls_t = cls_t.reshape(B, _NCLS, _R, _C)
    box_t = jnp.transpose(box_preds, (0, 2, 1))
    box_t = jnp.pad(box_t, ((0, 0), (0, 0), (0, padn))).reshape(B, 4, _R, _C)
    anc_t = jnp.pad(anchors.T, ((0, 0), (0, padn))).reshape(4, _R, _C)

    fields = _tc_decode(cls_t, box_t, anc_t, B).reshape(B * 12 * _NP)
    out = _sc_nms(fields).reshape(B, _MAXDET, 16)

    out_boxes = out[:, :, 7:11]
    out_classes = out[:, :, 11]
    out_conf = out[:, :, 12]
    return out_boxes, out_classes, out_conf
